# baseline (speedup 1.0000x reference)
import functools

import jax
import jax.numpy as jnp
from jax import lax
from jax.experimental import pallas as pl
from jax.experimental.pallas import tpu as pltpu
from jax.experimental.pallas import tpu_sc as plsc

B, N, VOCAB, EMB = 1024, 50, 1000, 64
B_BLK = 16
BT = 256
BSC = B - BT
LANE = 128
NV = VOCAB // 16


def _argmax_onehot(xb):
    bb = xb.shape[0]
    mx = jnp.max(xb, axis=2, keepdims=True)
    iota = lax.broadcasted_iota(jnp.int32, xb.shape, 2)
    idx = jnp.min(jnp.where(xb == mx, iota, VOCAB), axis=2)
    return (
        lax.broadcasted_iota(jnp.int32, (bb * N, VOCAB), 1)
        == idx.reshape(bb * N, 1)
    ).astype(jnp.float32)


def _tc1_block(x_ref, w_ref, out_ref):
    oh = _argmax_onehot(x_ref[...])
    res = jax.lax.dot(oh, w_ref[...], preferred_element_type=jnp.float32)
    out_ref[...] = res.reshape(B_BLK, N, EMB)


def _tc1(x, W):
    return pl.pallas_call(
        _tc1_block,
        grid=(BT // B_BLK,),
        in_specs=[
            pl.BlockSpec((B_BLK, N, VOCAB), lambda i: (i, 0, 0)),
            pl.BlockSpec((VOCAB, EMB), lambda i: (0, 0)),
        ],
        out_specs=pl.BlockSpec((B_BLK, N, EMB), lambda i: (i, 0, 0)),
        out_shape=jax.ShapeDtypeStruct((B, N, EMB), jnp.float32),
    )(x, W)


def _sc_tokens(x):
    info = plsc.get_sparse_core_info()
    nw = info.num_cores * info.num_subcores
    bpt = BSC // nw
    mesh = plsc.VectorSubcoreMesh(core_axis_name="c", subcore_axis_name="s")

    @functools.partial(
        pl.kernel,
        mesh=mesh,
        out_type=jax.ShapeDtypeStruct((BSC, LANE), jnp.int32),
        scratch_types=[
            pltpu.VMEM((2, N, VOCAB), jnp.float32),
            pltpu.VMEM((bpt, LANE), jnp.int32),
            pltpu.SemaphoreType.DMA,
            pltpu.SemaphoreType.DMA,
        ],
        compiler_params=pltpu.CompilerParams(needs_layout_passes=False),
    )
    def sc_argmax(x_hbm, tok_hbm, xbuf, tokbuf, d0, d1):
        wid = lax.axis_index("s") * info.num_cores + lax.axis_index("c")
        b0 = BT + wid * bpt
        lane = lax.broadcasted_iota(jnp.int32, (16,), 0)

        pltpu.async_copy(x_hbm.at[b0], xbuf.at[0], d0)
        pltpu.async_copy(x_hbm.at[b0 + 1], xbuf.at[1], d1)

        def batch_body(j, _):
            t = j & 1

            @pl.when(t == 0)
            def _():
                pltpu.make_async_copy(x_hbm.at[b0], xbuf.at[0], d0).wait()

            @pl.when(t == 1)
            def _():
                pltpu.make_async_copy(x_hbm.at[b0], xbuf.at[1], d1).wait()

            def token_body(n, _):
                m = xbuf[t, n, pl.ds(0, 16)]
                mi = lane
                for k in range(1, NV):
                    v = xbuf[t, n, pl.ds(k * 16, 16)]
                    gt = v > m
                    m = jnp.where(gt, v, m)
                    mi = jnp.where(gt, lane + (k * 16), mi)
                v = xbuf[t, n, pl.ds(VOCAB - 16, 16)]
                gt = v > m
                m = jnp.where(gt, v, m)
                mi = jnp.where(gt, lane + (VOCAB - 16), mi)
                gmax = jnp.max(m)
                tok = jnp.min(jnp.where(m == gmax, mi, VOCAB))
                plsc.store_scatter(
                    tokbuf,
                    [jnp.broadcast_to(j, (16,)), jnp.broadcast_to(n, (16,))],
                    jnp.broadcast_to(tok, (16,)),
                    mask=lane == 0,
                )
                return 0

            lax.fori_loop(0, N, token_body, 0)

            @pl.when((t == 0) & (j + 2 < bpt))
            def _():
                pltpu.async_copy(x_hbm.at[b0 + j + 2], xbuf.at[0], d0)

            @pl.when((t == 1) & (j + 2 < bpt))
            def _():
                pltpu.async_copy(x_hbm.at[b0 + j + 2], xbuf.at[1], d1)

            return 0

        lax.fori_loop(0, bpt, batch_body, 0)
        pltpu.sync_copy(tokbuf, tok_hbm.at[pl.ds(wid * bpt, bpt)])

    return sc_argmax(x)


def _tc2_block(tok_ref, w_ref, carry_ref, out_ref):
    del carry_ref
    idx = tok_ref[...][:, :N]
    oh = (
        lax.broadcasted_iota(jnp.int32, (B_BLK, N, VOCAB), 2)
        == idx[:, :, None]
    ).astype(jnp.float32).reshape(B_BLK * N, VOCAB)
    res = jax.lax.dot(oh, w_ref[...], preferred_element_type=jnp.float32)
    out_ref[...] = res.reshape(B_BLK, N, EMB)


def _tc2(toks, W, out_carry):
    return pl.pallas_call(
        _tc2_block,
        grid=(BSC // B_BLK,),
        in_specs=[
            pl.BlockSpec((B_BLK, LANE), lambda i: (i, 0)),
            pl.BlockSpec((VOCAB, EMB), lambda i: (0, 0)),
            pl.BlockSpec(memory_space=pltpu.MemorySpace.HBM),
        ],
        out_specs=pl.BlockSpec((B_BLK, N, EMB), lambda i: (BT // B_BLK + i, 0, 0)),
        out_shape=jax.ShapeDtypeStruct((B, N, EMB), jnp.float32),
        input_output_aliases={2: 0},
    )(toks, W, out_carry)


def kernel(x, W):
    out_tc = _tc1(x, W)
    toks = _sc_tokens(x)
    return _tc2(toks, W, out_tc)


# speedup vs baseline: 4.7975x; 4.7975x over previous
import jax
import jax.numpy as jnp
from jax import lax
from jax.experimental import pallas as pl

B, N, VOCAB, EMB = 1024, 50, 1000, 64
LG = 8


def _block(x_ref, w_ref, out_ref):
    xb = x_ref[0]
    mx = jnp.max(xb, axis=0, keepdims=True)
    iota = lax.broadcasted_iota(jnp.int32, (VOCAB, B), 0)
    idx = jnp.min(jnp.where(xb == mx, iota, VOCAB), axis=0)
    wt = w_ref[...]
    for lg in range(LG):
        oh = (
            lax.broadcasted_iota(jnp.int32, (VOCAB, B // LG), 0)
            == idx[None, lg * (B // LG) : (lg + 1) * (B // LG)]
        ).astype(jnp.float32)
        out_ref[0, :, lg * (B // LG) : (lg + 1) * (B // LG)] = jax.lax.dot(
            wt, oh, preferred_element_type=jnp.float32
        )


def kernel(x, W):
    xt = jnp.transpose(x, (1, 2, 0))
    wt = jnp.transpose(W, (1, 0))
    out_t = pl.pallas_call(
        _block,
        grid=(N,),
        in_specs=[
            pl.BlockSpec((1, VOCAB, B), lambda i: (i, 0, 0)),
            pl.BlockSpec((EMB, VOCAB), lambda i: (0, 0)),
        ],
        out_specs=pl.BlockSpec((1, EMB, B), lambda i: (i, 0, 0)),
        out_shape=jax.ShapeDtypeStruct((N, EMB, B), jnp.float32),
    )(xt, wt)
    return jnp.transpose(out_t, (2, 0, 1))
